# baseline (speedup 1.0000x reference)
import jax
import jax.numpy as jnp
from jax import lax
from jax.experimental import pallas as pl
from jax.experimental.pallas import tpu as pltpu
from jax.experimental.pallas import tpu_sc as plsc

_T = 50
_F = 48
_E = 16
_NPR = 4
_IW = _NPR * _T * _F
_OW = _NPR * _T * (_F + _E)
_NW = 32
_K = 2
_RUNS = _NPR * _T


def _sc_body(d_hbm, e_hbm, o_hbm, e_v, ibufs, obufs, esem, isems, osems, prows):
    rpw = prows // _NW
    nchunks = rpw // _K
    wid = lax.axis_index("s") * 2 + lax.axis_index("c")
    base = wid * rpw

    pltpu.make_async_copy(e_hbm, e_v, esem).start()
    pltpu.make_async_copy(e_hbm, e_v, esem).wait()

    for b in range(2):
        def _fill(r, _, buf=obufs[b]):
            t = lax.rem(r, _T)
            e = e_v[t, :]
            for c in range(_K):
                buf[c, pl.ds(r * (_F + _E) + _F, _E)] = e
            return 0
        lax.fori_loop(0, _RUNS, _fill, 0)

    def in_cp(b, i):
        return pltpu.make_async_copy(
            d_hbm.at[pl.ds(base + i * _K, _K)], ibufs[b], isems[b])

    def out_cp(b, i):
        return pltpu.make_async_copy(
            obufs[b], o_hbm.at[pl.ds(base + i * _K, _K)], osems[b])

    def assemble(b):
        src, dst = ibufs[b], obufs[b]

        def _runs(r, _):
            so = r * _F
            do = r * (_F + _E)
            for c in range(_K):
                for m in range(_F // 16):
                    dst[c, pl.ds(do + 16 * m, 16)] = src[c, pl.ds(so + 16 * m, 16)]
            return 0
        lax.fori_loop(0, _RUNS, _runs, 0)

    for b in range(2):
        in_cp(b, b).start()

    def _group(g, _):
        for b in range(2):
            i = 2 * g + b
            in_cp(b, i).wait()

            @pl.when(g > 0)
            def _():
                out_cp(b, i - 2).wait()
            assemble(b)
            out_cp(b, i).start()

            @pl.when(2 * g + b + 2 < nchunks)
            def _():
                in_cp(b, i + 2).start()
        return 0

    lax.fori_loop(0, nchunks // 2, _group, 0)

    for b in range(2):
        out_cp(b, nchunks - 2 + b).wait()


def kernel(data, embedding_weight):
    g, n, t, f = data.shape
    assert t == _T and f == _F and embedding_weight.shape == (_T, _E)
    prows = g * n // _NPR
    assert prows % (_NW * _K * 2) == 0

    d2 = data.reshape(prows, _IW)
    mesh = plsc.VectorSubcoreMesh(core_axis_name="c", subcore_axis_name="s")

    def body(d_ref, e_ref, o_ref, e_v, i0, i1, o0, o1, esem, s0, s1, q0, q1):
        _sc_body(d_ref, e_ref, o_ref, e_v, (i0, i1), (o0, o1), esem,
                 (s0, s1), (q0, q1), prows)

    run = pl.kernel(
        body,
        out_type=jax.ShapeDtypeStruct((prows, _OW), jnp.float32),
        mesh=mesh,
        scratch_types=[
            pltpu.VMEM((_T, _E), jnp.float32),
            pltpu.VMEM((_K, _IW), jnp.float32),
            pltpu.VMEM((_K, _IW), jnp.float32),
            pltpu.VMEM((_K, _OW), jnp.float32),
            pltpu.VMEM((_K, _OW), jnp.float32),
            pltpu.SemaphoreType.DMA,
            pltpu.SemaphoreType.DMA,
            pltpu.SemaphoreType.DMA,
            pltpu.SemaphoreType.DMA,
            pltpu.SemaphoreType.DMA,
        ],
    )
    out = run(d2, embedding_weight)
    return out.reshape(g, n, t, _F + _E)


# speedup vs baseline: 1.0037x; 1.0037x over previous
import jax
import jax.numpy as jnp
from jax import lax
from jax.experimental import pallas as pl
from jax.experimental.pallas import tpu as pltpu
from jax.experimental.pallas import tpu_sc as plsc

_T = 50
_F = 48
_E = 16
_NPR = 4
_IW = _NPR * _T * _F
_OW = _NPR * _T * (_F + _E)
_NW = 32
_K = 2
_RUNS = _NPR * _T


def _sc_body(d_hbm, e_hbm, o_hbm, e_v, ibufs, obufs, esem, isems, osems, prows):
    rpw = prows // _NW
    nchunks = rpw // _K
    wid = lax.axis_index("s") * 2 + lax.axis_index("c")
    base = wid * rpw

    pltpu.make_async_copy(e_hbm, e_v, esem).start()
    pltpu.make_async_copy(e_hbm, e_v, esem).wait()

    for b in range(2):
        def _fill(r, _, buf=obufs[b]):
            t = lax.rem(r, _T)
            e = e_v[t, :]
            for c in range(_K):
                buf[c, pl.ds(r * (_F + _E) + _F, _E)] = e
            return 0
        lax.fori_loop(0, _RUNS, _fill, 0)

    def in_cp(b, i):
        return pltpu.make_async_copy(
            d_hbm.at[pl.ds(base + i * _K, _K)], ibufs[b], isems[b])

    def out_cp(b, i):
        return pltpu.make_async_copy(
            obufs[b], o_hbm.at[pl.ds(base + i * _K, _K)], osems[b])

    def assemble(b):
        src, dst = ibufs[b], obufs[b]
        unroll = 8

        def _runs(q, _):
            so = q * (unroll * _F)
            do = q * (unroll * (_F + _E))
            for c in range(_K):
                for rr in range(unroll):
                    for m in range(_F // 16):
                        dst[c, pl.ds(do + rr * (_F + _E) + 16 * m, 16)] = (
                            src[c, pl.ds(so + rr * _F + 16 * m, 16)])
            return 0
        lax.fori_loop(0, _RUNS // unroll, _runs, 0)

    for b in range(2):
        in_cp(b, b).start()

    def _group(g, _):
        for b in range(2):
            i = 2 * g + b
            in_cp(b, i).wait()

            @pl.when(g > 0)
            def _():
                out_cp(b, i - 2).wait()
            assemble(b)
            out_cp(b, i).start()

            @pl.when(2 * g + b + 2 < nchunks)
            def _():
                in_cp(b, i + 2).start()
        return 0

    lax.fori_loop(0, nchunks // 2, _group, 0)

    for b in range(2):
        out_cp(b, nchunks - 2 + b).wait()


def kernel(data, embedding_weight):
    g, n, t, f = data.shape
    assert t == _T and f == _F and embedding_weight.shape == (_T, _E)
    prows = g * n // _NPR
    assert prows % (_NW * _K * 2) == 0

    d2 = data.reshape(prows, _IW)
    mesh = plsc.VectorSubcoreMesh(core_axis_name="c", subcore_axis_name="s")

    def body(d_ref, e_ref, o_ref, e_v, i0, i1, o0, o1, esem, s0, s1, q0, q1):
        _sc_body(d_ref, e_ref, o_ref, e_v, (i0, i1), (o0, o1), esem,
                 (s0, s1), (q0, q1), prows)

    run = pl.kernel(
        body,
        out_type=jax.ShapeDtypeStruct((prows, _OW), jnp.float32),
        mesh=mesh,
        scratch_types=[
            pltpu.VMEM((_T, _E), jnp.float32),
            pltpu.VMEM((_K, _IW), jnp.float32),
            pltpu.VMEM((_K, _IW), jnp.float32),
            pltpu.VMEM((_K, _OW), jnp.float32),
            pltpu.VMEM((_K, _OW), jnp.float32),
            pltpu.SemaphoreType.DMA,
            pltpu.SemaphoreType.DMA,
            pltpu.SemaphoreType.DMA,
            pltpu.SemaphoreType.DMA,
            pltpu.SemaphoreType.DMA,
        ],
    )
    out = run(d2, embedding_weight)
    return out.reshape(g, n, t, _F + _E)


# speedup vs baseline: 1.0905x; 1.0865x over previous
import jax
import jax.numpy as jnp
from jax.experimental import pallas as pl

_T = 50
_F = 48
_E = 16
_NPR = 4
_IW = _NPR * _T * _F
_OW = _NPR * _T * (_F + _E)
_B = 64


def _body(d_ref, p_ref, o_ref):
    b = d_ref.shape[0]
    p = p_ref[:, :]
    for j in range(_OW // 128):
        o_ref[:, 128 * j : 128 * j + _F] = d_ref[:, 96 * j : 96 * j + _F]
        o_ref[:, 128 * j + _F : 128 * j + 64] = jnp.broadcast_to(
            p[j : j + 1, 0:_E], (b, _E))
        o_ref[:, 128 * j + 64 : 128 * j + 64 + _F] = (
            d_ref[:, 96 * j + _F : 96 * j + 2 * _F])
        o_ref[:, 128 * j + 64 + _F : 128 * (j + 1)] = jnp.broadcast_to(
            p[j : j + 1, _E : 2 * _E], (b, _E))


def kernel(data, embedding_weight):
    g, n, t, f = data.shape
    assert t == _T and f == _F and embedding_weight.shape == (_T, _E)
    prows = g * n // _NPR
    assert prows % _B == 0

    d2 = data.reshape(prows, _IW)
    rt = jnp.arange(2 * (_OW // 128), dtype=jnp.int32) % _T
    pat = embedding_weight[rt].reshape(_OW // 128, 2 * _E)

    out = pl.pallas_call(
        _body,
        grid=(prows // _B,),
        in_specs=[
            pl.BlockSpec((_B, _IW), lambda i: (i, 0)),
            pl.BlockSpec((_OW // 128, 2 * _E), lambda i: (0, 0)),
        ],
        out_specs=pl.BlockSpec((_B, _OW), lambda i: (i, 0)),
        out_shape=jax.ShapeDtypeStruct((prows, _OW), data.dtype),
    )(d2, pat)
    return out.reshape(g, n, t, _F + _E)


# speedup vs baseline: 9.1765x; 8.4146x over previous
import jax
import jax.numpy as jnp
from jax.experimental import pallas as pl

_T = 50
_F = 48
_E = 16


_TB = 5


def _body(d_ref, e_ref, o_ref):
    for tt in range(_TB):
        o_ref[0, tt, :_F, :] = d_ref[0, tt, :, :]
        o_ref[0, tt, _F:, :] = e_ref[tt * _E : (tt + 1) * _E, :]


def kernel(data, embedding_weight):
    g, n, t, f = data.shape
    assert t == _T and f == _F and embedding_weight.shape == (_T, _E)

    d_t = jnp.transpose(data, (0, 2, 3, 1))
    eb = jnp.broadcast_to(embedding_weight.reshape(t * _E, 1), (t * _E, n))

    out_t = pl.pallas_call(
        _body,
        grid=(g, t // _TB),
        in_specs=[
            pl.BlockSpec((1, _TB, f, n), lambda i, j: (i, j, 0, 0)),
            pl.BlockSpec((_TB * _E, n), lambda i, j: (j, 0)),
        ],
        out_specs=pl.BlockSpec((1, _TB, f + _E, n), lambda i, j: (i, j, 0, 0)),
        out_shape=jax.ShapeDtypeStruct((g, t, f + _E, n), data.dtype),
    )(d_t, eb)
    return jnp.transpose(out_t, (0, 3, 1, 2))


# speedup vs baseline: 9.6175x; 1.0481x over previous
import jax
import jax.numpy as jnp
from jax.experimental import pallas as pl

_T = 50
_F = 48
_E = 16


_TB = 10


def _body(d_ref, e_ref, o_ref):
    for tt in range(_TB):
        o_ref[0, tt, :_F, :] = d_ref[0, tt, :, :]
        o_ref[0, tt, _F:, :] = e_ref[tt * _E : (tt + 1) * _E, :]


def kernel(data, embedding_weight):
    g, n, t, f = data.shape
    assert t == _T and f == _F and embedding_weight.shape == (_T, _E)

    d_t = jnp.transpose(data, (0, 2, 3, 1))
    eb = jnp.broadcast_to(embedding_weight.reshape(t * _E, 1), (t * _E, n))

    out_t = pl.pallas_call(
        _body,
        grid=(g, t // _TB),
        in_specs=[
            pl.BlockSpec((1, _TB, f, n), lambda i, j: (i, j, 0, 0)),
            pl.BlockSpec((_TB * _E, n), lambda i, j: (j, 0)),
        ],
        out_specs=pl.BlockSpec((1, _TB, f + _E, n), lambda i, j: (i, j, 0, 0)),
        out_shape=jax.ShapeDtypeStruct((g, t, f + _E, n), data.dtype),
    )(d_t, eb)
    return jnp.transpose(out_t, (0, 3, 1, 2))
